# baseline (speedup 1.0000x reference)
import jax
import jax.numpy as jnp
from jax.experimental import pallas as pl
from jax.experimental.pallas import tpu as pltpu
from jax.experimental.pallas import tpu_sc as plsc


_LANES = 16
_BM = 8


def _tc_add_kernel(x_ref, pe_ref, o_ref):
    o_ref[...] = x_ref[...] + pe_ref[...][None]


def _tc_kernel(x, pe_weight):
    B, S, D = x.shape
    BS = 512
    return pl.pallas_call(
        _tc_add_kernel,
        grid=(S // BS,),
        in_specs=[
            pl.BlockSpec((B, BS, D), lambda s: (0, s, 0)),
            pl.BlockSpec((BS, D), lambda s: (s, 0)),
        ],
        out_specs=pl.BlockSpec((B, BS, D), lambda s: (0, s, 0)),
        out_shape=jax.ShapeDtypeStruct((B, S, D), x.dtype),
    )(x, pe_weight[:S])


def _sc_kernel(x2d, pe):
    R, D = x2d.shape
    S = pe.shape[0]
    n_pe_blocks = S // _BM

    mesh = plsc.VectorSubcoreMesh(core_axis_name="core",
                                  subcore_axis_name="subcore")

    @pl.kernel(out_type=jax.ShapeDtypeStruct((R, D), x2d.dtype), mesh=mesh,
               scratch_types=[])
    def sc_add(x_hbm, pe_hbm, o_hbm):
        def body(x_vmem, pe_vmem, o_vmem):
            @pl.loop(0, _BM)
            def _(r):
                @pl.loop(0, D, step=_LANES)
                def _(c):
                    slc = (pl.ds(r, 1), pl.ds(c, _LANES))
                    o_vmem.at[*slc][...] = (
                        x_vmem.at[*slc][...] + pe_vmem.at[*slc][...]
                    )

        pltpu.emit_pipeline(
            body,
            grid=(R // _BM,),
            in_specs=[
                pl.BlockSpec((_BM, D), index_map=lambda i: (i, 0)),
                pl.BlockSpec((_BM, D), index_map=lambda i: (i % n_pe_blocks, 0)),
            ],
            out_specs=[pl.BlockSpec((_BM, D), index_map=lambda i: (i, 0))],
            core_axis_name=("core", "subcore"),
            dimension_semantics=(pltpu.PARALLEL,),
        )(x_hbm, pe_hbm, o_hbm)

    return sc_add(x2d, pe)


def kernel(x, pe_weight):
    B, S, D = x.shape
    out2d = _sc_kernel(x.reshape(B * S, D), pe_weight[:S])
    return out2d.reshape(B, S, D)


# speedup vs baseline: 3.9425x; 3.9425x over previous
import jax
import jax.numpy as jnp
from jax.experimental import pallas as pl
from jax.experimental.pallas import tpu as pltpu
from jax.experimental.pallas import tpu_sc as plsc


_LANES = 16
_BM = 8


def _tc_add_kernel(x_ref, pe_ref, o_ref):
    o_ref[...] = x_ref[...] + pe_ref[...][None]


def _tc_kernel(x, pe_weight):
    B, S, D = x.shape
    BS = 512
    return pl.pallas_call(
        _tc_add_kernel,
        grid=(S // BS,),
        in_specs=[
            pl.BlockSpec((B, BS, D), lambda s: (0, s, 0)),
            pl.BlockSpec((BS, D), lambda s: (s, 0)),
        ],
        out_specs=pl.BlockSpec((B, BS, D), lambda s: (0, s, 0)),
        out_shape=jax.ShapeDtypeStruct((B, S, D), x.dtype),
    )(x, pe_weight[:S])


def _sc_kernel(x2d, pe):
    R, D = x2d.shape
    S = pe.shape[0]
    n_pe_blocks = S // _BM

    mesh = plsc.VectorSubcoreMesh(core_axis_name="core",
                                  subcore_axis_name="subcore")

    @pl.kernel(out_type=jax.ShapeDtypeStruct((R, D), x2d.dtype), mesh=mesh,
               scratch_types=[])
    def sc_add(x_hbm, pe_hbm, o_hbm):
        def body(x_vmem, pe_vmem, o_vmem):
            @pl.loop(0, _BM)
            def _(r):
                @pl.loop(0, D, step=_LANES)
                def _(c):
                    slc = (pl.ds(r, 1), pl.ds(c, _LANES))
                    o_vmem.at[*slc][...] = (
                        x_vmem.at[*slc][...] + pe_vmem.at[*slc][...]
                    )

        pltpu.emit_pipeline(
            body,
            grid=(R // _BM,),
            in_specs=[
                pl.BlockSpec((_BM, D), index_map=lambda i: (i, 0)),
                pl.BlockSpec((_BM, D), index_map=lambda i: (i % n_pe_blocks, 0)),
            ],
            out_specs=[pl.BlockSpec((_BM, D), index_map=lambda i: (i, 0))],
            core_axis_name=("core", "subcore"),
            dimension_semantics=(pltpu.PARALLEL,),
        )(x_hbm, pe_hbm, o_hbm)

    return sc_add(x2d, pe)


def kernel(x, pe_weight):
    return _tc_kernel(x, pe_weight)
